# baseline (speedup 1.0000x reference)
import jax
import jax.numpy as jnp
from jax import lax
from jax.experimental import pallas as pl
from jax.experimental.pallas import tpu as pltpu
from jax.experimental.pallas import tpu_sc as plsc

H = 128
N = 10000
NP = 10240
E = 320000
NC = 2
NS = 16
RPT = NP // NS
BI = 16

CA = 160
NBA = CA // BI
EPC_A = CA * 128 * NS

CB = 80
NBB = CB // BI
EB = CB * 128 * (NC * NS)


def _pipelined_blocks(x_hbm, src_hbm, dst_hbm, row0, nblocks,
                      acc, sbuf, dbuf, brow, rows, gsem, ssem,
                      cacc=None, ones=None, osem=None):
    def unpack_half(b, h):
        def row_body(r, carry):
            for j in range(4):
                v = brow[b, 64 * h + r, pl.ds(16 * j, 16)]
                rows[h, r, pl.ds(32 * j, 16)] = lax.bitcast_convert_type(
                    v << 16, jnp.float32)
                rows[h, r, pl.ds(32 * j + 16, 16)] = lax.bitcast_convert_type(
                    v & jnp.int32(-65536), jnp.float32)
            return carry
        lax.fori_loop(0, 64, row_body, 0)

    def outer(o, carry):
        pltpu.sync_copy(src_hbm.at[pl.ds(row0 + o * BI, BI)], sbuf)
        pltpu.sync_copy(dst_hbm.at[pl.ds(row0 + o * BI, BI)], dbuf)
        scat = [None, None]
        cscat = [None, None]
        gd = [[None, None], [None, None]]

        def issue(g, buf):
            for hh in range(2):
                gd[buf][hh] = pltpu.async_copy(
                    x_hbm.at[sbuf.at[g, pl.ds(64 * hh, 64)]],
                    brow.at[buf, pl.ds(64 * hh, 64)], gsem.at[buf, hh])

        issue(0, 0)
        for g in range(BI):
            b = g & 1
            nb = 1 - b
            if g + 1 < BI:
                issue(g + 1, nb)
            if cacc is not None:
                if cscat[b] is not None:
                    cscat[b].wait()
                cscat[b] = pltpu.async_copy(ones, cacc.at[dbuf.at[g]], osem.at[b],
                                            add=True)
            for h in range(2):
                gd[b][h].wait()
                if scat[h] is not None:
                    scat[h].wait()
                unpack_half(b, h)
                scat[h] = pltpu.async_copy(
                    rows.at[h], acc.at[dbuf.at[g, pl.ds(64 * h, 64)]],
                    ssem.at[h], add=True)
        scat[0].wait()
        scat[1].wait()
        if cacc is not None:
            cscat[0].wait()
            cscat[1].wait()
        return carry

    lax.fori_loop(0, nblocks, outer, 0)


def _sc_layer0_body(x_hbm, src_hbm, dst_hbm, zeros_hbm, agg_out, cnt_out,
                    acc, cacc, sbuf, dbuf, brow, rows, ones, zb, gsem, ssem, osem):
    cid = lax.axis_index("c")
    sid = lax.axis_index("s")
    r0 = sid * RPT
    pltpu.sync_copy(zeros_hbm.at[pl.ds(r0, RPT)], acc.at[pl.ds(r0, RPT)])
    for i in range(RPT // 16):
        zb[pl.ds(i * 16, 16)] = jnp.zeros((16,), jnp.float32)
    pltpu.sync_copy(zb, cacc.at[pl.ds(r0, RPT)])
    for i in range(8):
        ones[pl.ds(i * 16, 16)] = jnp.ones((16,), jnp.float32)
    row0 = cid * (EPC_A // 128) + sid * CA
    plsc.subcore_barrier()
    _pipelined_blocks(x_hbm, src_hbm, dst_hbm, row0, NBA,
                      acc, sbuf, dbuf, brow, rows, gsem, ssem,
                      cacc=cacc, ones=ones, osem=osem)
    plsc.subcore_barrier()
    pltpu.sync_copy(acc.at[pl.ds(r0, RPT)], agg_out.at[cid, pl.ds(r0, RPT)])
    pltpu.sync_copy(cacc.at[pl.ds(r0, RPT)], cnt_out.at[cid, pl.ds(r0, RPT)])


def _sc_layer1_body(x_hbm, src_hbm, dst_hbm, zeros_hbm, agg_out,
                    acc, sbuf, dbuf, brow, rows, gsem, ssem):
    cid = lax.axis_index("c")
    sid = lax.axis_index("s")
    r0 = sid * RPT
    pltpu.sync_copy(zeros_hbm.at[pl.ds(r0, RPT)], acc.at[pl.ds(r0, RPT)])
    row0 = (cid * NS + sid) * CB
    plsc.subcore_barrier()
    _pipelined_blocks(x_hbm, src_hbm, dst_hbm, row0, NBB,
                      acc, sbuf, dbuf, brow, rows, gsem, ssem)
    plsc.subcore_barrier()
    pltpu.sync_copy(acc.at[pl.ds(r0, RPT)], agg_out.at[cid, pl.ds(r0, RPT)])


_sc_mesh = plsc.VectorSubcoreMesh(core_axis_name="c", subcore_axis_name="s")
_sc_params = pltpu.CompilerParams(use_tc_tiling_on_sc=False)

_sc_layer0 = pl.kernel(
    _sc_layer0_body,
    compiler_params=_sc_params,
    out_type=(jax.ShapeDtypeStruct((NC, NP, H), jnp.float32),
              jax.ShapeDtypeStruct((NC, NP), jnp.float32)),
    mesh=_sc_mesh,
    scratch_types=[
        pltpu.VMEM_SHARED((NP, H), jnp.float32),
        pltpu.VMEM_SHARED((NP,), jnp.float32),
        pltpu.VMEM((BI, 128), jnp.int32),
        pltpu.VMEM((BI, 128), jnp.int32),
        pltpu.VMEM((2, 128, H // 2), jnp.int32),
        pltpu.VMEM((2, 64, H), jnp.float32),
        pltpu.VMEM((128,), jnp.float32),
        pltpu.VMEM((RPT,), jnp.float32),
        pltpu.SemaphoreType.DMA((2, 2)),
        pltpu.SemaphoreType.DMA((2,)),
        pltpu.SemaphoreType.DMA((2,)),
    ],
)

_sc_layer1 = pl.kernel(
    _sc_layer1_body,
    compiler_params=_sc_params,
    out_type=jax.ShapeDtypeStruct((NC, NP, H), jnp.float32),
    mesh=_sc_mesh,
    scratch_types=[
        pltpu.VMEM_SHARED((NP, H), jnp.float32),
        pltpu.VMEM((BI, 128), jnp.int32),
        pltpu.VMEM((BI, 128), jnp.int32),
        pltpu.VMEM((2, 128, H // 2), jnp.int32),
        pltpu.VMEM((2, 64, H), jnp.float32),
        pltpu.SemaphoreType.DMA((2, 2)),
        pltpu.SemaphoreType.DMA((2,)),
    ],
)

R = 1000


def _tc_l0_body(agg_ref, cnt_ref, xs0_ref, xt0_ref,
                wl_st, wr_st, wl_ts, wr_ts, bl_st, bl_ts,
                xs1_ref, xt1_ref):
    meanT = agg_ref[0] / jnp.maximum(cnt_ref[0], 1.0)
    meanS = agg_ref[1] / jnp.maximum(cnt_ref[1], 1.0)
    xt = (jnp.dot(meanT, wl_st[...], preferred_element_type=jnp.float32)
          + bl_st[...]
          + jnp.dot(xt0_ref[...], wr_st[...], preferred_element_type=jnp.float32))
    xs = (jnp.dot(meanS, wl_ts[...], preferred_element_type=jnp.float32)
          + bl_ts[...]
          + jnp.dot(xs0_ref[...], wr_ts[...], preferred_element_type=jnp.float32))
    xt1_ref[...] = jnp.where(xt >= 0, xt, 0.01 * xt)
    xs1_ref[...] = jnp.where(xs >= 0, xs, 0.01 * xs)


def _tc_l1_body(agg_ref, cnt_ref, xs1_ref, wl, wr, linw, bl, linb, out_ref):
    mean = (agg_ref[0] + agg_ref[1]) / jnp.maximum(cnt_ref[...], 1.0)
    xs = (jnp.dot(mean, wl[...], preferred_element_type=jnp.float32)
          + bl[...]
          + jnp.dot(xs1_ref[...], wr[...], preferred_element_type=jnp.float32))
    xs = jnp.where(xs >= 0, xs, 0.01 * xs)
    out_ref[...] = jnp.dot(xs, linw[...], preferred_element_type=jnp.float32) + linb[...]


def _w_spec():
    return pl.BlockSpec((H, H), lambda i: (0, 0))


def _b_spec():
    return pl.BlockSpec((1, H), lambda i: (0, 0))


_tc_l0 = pl.pallas_call(
    _tc_l0_body,
    grid=(N // R,),
    in_specs=[
        pl.BlockSpec((NC, R, H), lambda i: (0, i, 0)),
        pl.BlockSpec((NC, R, 1), lambda i: (0, i, 0)),
        pl.BlockSpec((R, H), lambda i: (i, 0)),
        pl.BlockSpec((R, H), lambda i: (i, 0)),
        _w_spec(), _w_spec(), _w_spec(), _w_spec(), _b_spec(), _b_spec(),
    ],
    out_specs=(pl.BlockSpec((R, H), lambda i: (i, 0)),
               pl.BlockSpec((R, H), lambda i: (i, 0))),
    out_shape=(jax.ShapeDtypeStruct((N, H), jnp.float32),
               jax.ShapeDtypeStruct((N, H), jnp.float32)),
)

_tc_l1 = pl.pallas_call(
    _tc_l1_body,
    grid=(N // R,),
    in_specs=[
        pl.BlockSpec((NC, R, H), lambda i: (0, i, 0)),
        pl.BlockSpec((R, 1), lambda i: (i, 0)),
        pl.BlockSpec((R, H), lambda i: (i, 0)),
        _w_spec(), _w_spec(), _w_spec(), _b_spec(), _b_spec(),
    ],
    out_specs=pl.BlockSpec((R, H), lambda i: (i, 0)),
    out_shape=jax.ShapeDtypeStruct((N, H), jnp.float32),
)


def _pack_bf16(x):
    xp = x.reshape(-1, 4, 2, 16).transpose(0, 1, 3, 2).reshape(-1, 128)
    xb = xp.astype(jnp.bfloat16)
    return lax.bitcast_convert_type(xb.reshape(-1, 64, 2), jnp.int32)


def kernel(x_source, x_target, edge_index_st, edge_index_ts,
           l0_st_Wl, l0_st_bl, l0_st_Wr, l0_ts_Wl, l0_ts_bl, l0_ts_Wr,
           l1_st_Wl, l1_st_bl, l1_st_Wr, l1_ts_Wl, l1_ts_bl, l1_ts_Wr,
           lin_W, lin_b):
    f32 = jnp.float32
    xp_cat = _pack_bf16(jnp.concatenate([x_source, x_target], axis=0))

    s_st = edge_index_st[0].astype(jnp.int32)
    d_st = edge_index_st[1].astype(jnp.int32)
    s_ts = edge_index_ts[0].astype(jnp.int32)
    d_ts = edge_index_ts[1].astype(jnp.int32)

    padA = EPC_A - E
    pad_src = (jnp.arange(padA, dtype=jnp.int32) * 53) % N
    pad_dst = N + jnp.arange(padA, dtype=jnp.int32) % (NP - N)
    srcA = jnp.concatenate([
        s_st, pad_src,
        s_ts + N, pad_src,
    ]).reshape(-1, 128)
    dstA = jnp.concatenate([
        d_st, pad_dst,
        d_ts, pad_dst,
    ]).reshape(-1, 128)

    zeros_big = jnp.zeros((NP, H), f32)
    agg0, cnt0 = _sc_layer0(xp_cat, srcA, dstA, zeros_big)
    cntb = cnt0[:, :N, None]

    xs1, xt1 = _tc_l0(agg0, cntb, x_source, x_target,
                      l0_st_Wl, l0_st_Wr, l0_ts_Wl, l0_ts_Wr,
                      l0_st_bl.reshape(1, H), l0_ts_bl.reshape(1, H))

    padB = EB - E
    srcB = jnp.concatenate([s_ts, pad_src]).reshape(-1, 128)
    dstB = jnp.concatenate([d_ts, pad_dst]).reshape(-1, 128)
    agg1 = _sc_layer1(_pack_bf16(xt1), srcB, dstB, zeros_big)

    return _tc_l1(agg1, cntb[1], xs1,
                  l1_ts_Wl, l1_ts_Wr, lin_W,
                  l1_ts_bl.reshape(1, H), lin_b.reshape(1, H))


# speedup vs baseline: 1.0725x; 1.0725x over previous
import jax
import jax.numpy as jnp
from jax import lax
from jax.experimental import pallas as pl
from jax.experimental.pallas import tpu as pltpu
from jax.experimental.pallas import tpu_sc as plsc

H = 128
N = 10000
NP = 10240
E = 320000
NC = 2
NS = 16
RPT = NP // NS
BI = 16

CA = 160
NBA = CA // BI
EPC_A = CA * 128 * NS

CB = 80
NBB = CB // BI
EB = CB * 128 * (NC * NS)


def _pipelined_blocks(x_hbm, src_hbm, dst_hbm, row0, nblocks,
                      acc, sbuf, dbuf, brow, rows, gsem, ssem,
                      cacc=None, ones=None, osem=None):
    def unpack_half(b, h):
        def row_body(r, carry):
            for j in range(4):
                v = brow[b, 64 * h + r, pl.ds(16 * j, 16)]
                rows[h, r, pl.ds(32 * j, 16)] = lax.bitcast_convert_type(
                    v << 16, jnp.float32)
                rows[h, r, pl.ds(32 * j + 16, 16)] = lax.bitcast_convert_type(
                    v, jnp.float32)
            return carry
        lax.fori_loop(0, 64, row_body, 0)

    def outer(o, carry):
        pltpu.sync_copy(src_hbm.at[pl.ds(row0 + o * BI, BI)], sbuf)
        pltpu.sync_copy(dst_hbm.at[pl.ds(row0 + o * BI, BI)], dbuf)
        scat = [None, None]
        cscat = [None, None]
        gd = [[None, None], [None, None]]

        def issue(g, buf):
            for hh in range(2):
                gd[buf][hh] = pltpu.async_copy(
                    x_hbm.at[sbuf.at[g, pl.ds(64 * hh, 64)]],
                    brow.at[buf, pl.ds(64 * hh, 64)], gsem.at[buf, hh])

        issue(0, 0)
        for g in range(BI):
            b = g & 1
            nb = 1 - b
            if g + 1 < BI:
                issue(g + 1, nb)
            if cacc is not None:
                if cscat[b] is not None:
                    cscat[b].wait()
                cscat[b] = pltpu.async_copy(ones, cacc.at[dbuf.at[g]], osem.at[b],
                                            add=True)
            for h in range(2):
                gd[b][h].wait()
                if scat[h] is not None:
                    scat[h].wait()
                unpack_half(b, h)
                scat[h] = pltpu.async_copy(
                    rows.at[h], acc.at[dbuf.at[g, pl.ds(64 * h, 64)]],
                    ssem.at[h], add=True)
        scat[0].wait()
        scat[1].wait()
        if cacc is not None:
            cscat[0].wait()
            cscat[1].wait()
        return carry

    lax.fori_loop(0, nblocks, outer, 0)


def _sc_layer0_body(x_hbm, src_hbm, dst_hbm, zeros_hbm, agg_out, cnt_out,
                    acc, cacc, sbuf, dbuf, brow, rows, ones, zb, gsem, ssem, osem):
    cid = lax.axis_index("c")
    sid = lax.axis_index("s")
    r0 = sid * RPT
    pltpu.sync_copy(zeros_hbm.at[pl.ds(r0, RPT)], acc.at[pl.ds(r0, RPT)])
    for i in range(RPT // 16):
        zb[pl.ds(i * 16, 16)] = jnp.zeros((16,), jnp.float32)
    pltpu.sync_copy(zb, cacc.at[pl.ds(r0, RPT)])
    for i in range(8):
        ones[pl.ds(i * 16, 16)] = jnp.ones((16,), jnp.float32)
    row0 = cid * (EPC_A // 128) + sid * CA
    plsc.subcore_barrier()
    _pipelined_blocks(x_hbm, src_hbm, dst_hbm, row0, NBA,
                      acc, sbuf, dbuf, brow, rows, gsem, ssem,
                      cacc=cacc, ones=ones, osem=osem)
    plsc.subcore_barrier()
    pltpu.sync_copy(acc.at[pl.ds(r0, RPT)], agg_out.at[cid, pl.ds(r0, RPT)])
    pltpu.sync_copy(cacc.at[pl.ds(r0, RPT)], cnt_out.at[cid, pl.ds(r0, RPT)])


def _sc_layer1_body(x_hbm, src_hbm, dst_hbm, zeros_hbm, agg_out,
                    acc, sbuf, dbuf, brow, rows, gsem, ssem):
    cid = lax.axis_index("c")
    sid = lax.axis_index("s")
    r0 = sid * RPT
    pltpu.sync_copy(zeros_hbm.at[pl.ds(r0, RPT)], acc.at[pl.ds(r0, RPT)])
    row0 = (cid * NS + sid) * CB
    plsc.subcore_barrier()
    _pipelined_blocks(x_hbm, src_hbm, dst_hbm, row0, NBB,
                      acc, sbuf, dbuf, brow, rows, gsem, ssem)
    plsc.subcore_barrier()
    pltpu.sync_copy(acc.at[pl.ds(r0, RPT)], agg_out.at[cid, pl.ds(r0, RPT)])


_sc_mesh = plsc.VectorSubcoreMesh(core_axis_name="c", subcore_axis_name="s")
_sc_params = pltpu.CompilerParams(use_tc_tiling_on_sc=False)

_sc_layer0 = pl.kernel(
    _sc_layer0_body,
    compiler_params=_sc_params,
    out_type=(jax.ShapeDtypeStruct((NC, NP, H), jnp.float32),
              jax.ShapeDtypeStruct((NC, NP), jnp.float32)),
    mesh=_sc_mesh,
    scratch_types=[
        pltpu.VMEM_SHARED((NP, H), jnp.float32),
        pltpu.VMEM_SHARED((NP,), jnp.float32),
        pltpu.VMEM((BI, 128), jnp.int32),
        pltpu.VMEM((BI, 128), jnp.int32),
        pltpu.VMEM((2, 128, H // 2), jnp.int32),
        pltpu.VMEM((2, 64, H), jnp.float32),
        pltpu.VMEM((128,), jnp.float32),
        pltpu.VMEM((RPT,), jnp.float32),
        pltpu.SemaphoreType.DMA((2, 2)),
        pltpu.SemaphoreType.DMA((2,)),
        pltpu.SemaphoreType.DMA((2,)),
    ],
)

_sc_layer1 = pl.kernel(
    _sc_layer1_body,
    compiler_params=_sc_params,
    out_type=jax.ShapeDtypeStruct((NC, NP, H), jnp.float32),
    mesh=_sc_mesh,
    scratch_types=[
        pltpu.VMEM_SHARED((NP, H), jnp.float32),
        pltpu.VMEM((BI, 128), jnp.int32),
        pltpu.VMEM((BI, 128), jnp.int32),
        pltpu.VMEM((2, 128, H // 2), jnp.int32),
        pltpu.VMEM((2, 64, H), jnp.float32),
        pltpu.SemaphoreType.DMA((2, 2)),
        pltpu.SemaphoreType.DMA((2,)),
    ],
)

R = 1000


def _tc_l0_body(agg_ref, cnt_ref, xs0_ref, xt0_ref,
                wl_st, wr_st, wl_ts, wr_ts, bl_st, bl_ts,
                xs1_ref, xt1_ref):
    meanT = agg_ref[0] / jnp.maximum(cnt_ref[0], 1.0)
    meanS = agg_ref[1] / jnp.maximum(cnt_ref[1], 1.0)
    xt = (jnp.dot(meanT, wl_st[...], preferred_element_type=jnp.float32)
          + bl_st[...]
          + jnp.dot(xt0_ref[...], wr_st[...], preferred_element_type=jnp.float32))
    xs = (jnp.dot(meanS, wl_ts[...], preferred_element_type=jnp.float32)
          + bl_ts[...]
          + jnp.dot(xs0_ref[...], wr_ts[...], preferred_element_type=jnp.float32))
    xt1_ref[...] = jnp.where(xt >= 0, xt, 0.01 * xt)
    xs1_ref[...] = jnp.where(xs >= 0, xs, 0.01 * xs)


def _tc_l1_body(agg_ref, cnt_ref, xs1_ref, wl, wr, linw, bl, linb, out_ref):
    mean = (agg_ref[0] + agg_ref[1]) / jnp.maximum(cnt_ref[...], 1.0)
    xs = (jnp.dot(mean, wl[...], preferred_element_type=jnp.float32)
          + bl[...]
          + jnp.dot(xs1_ref[...], wr[...], preferred_element_type=jnp.float32))
    xs = jnp.where(xs >= 0, xs, 0.01 * xs)
    out_ref[...] = jnp.dot(xs, linw[...], preferred_element_type=jnp.float32) + linb[...]


def _w_spec():
    return pl.BlockSpec((H, H), lambda i: (0, 0))


def _b_spec():
    return pl.BlockSpec((1, H), lambda i: (0, 0))


_tc_l0 = pl.pallas_call(
    _tc_l0_body,
    grid=(N // R,),
    in_specs=[
        pl.BlockSpec((NC, R, H), lambda i: (0, i, 0)),
        pl.BlockSpec((NC, R, 1), lambda i: (0, i, 0)),
        pl.BlockSpec((R, H), lambda i: (i, 0)),
        pl.BlockSpec((R, H), lambda i: (i, 0)),
        _w_spec(), _w_spec(), _w_spec(), _w_spec(), _b_spec(), _b_spec(),
    ],
    out_specs=(pl.BlockSpec((R, H), lambda i: (i, 0)),
               pl.BlockSpec((R, H), lambda i: (i, 0))),
    out_shape=(jax.ShapeDtypeStruct((N, H), jnp.float32),
               jax.ShapeDtypeStruct((N, H), jnp.float32)),
)

_tc_l1 = pl.pallas_call(
    _tc_l1_body,
    grid=(N // R,),
    in_specs=[
        pl.BlockSpec((NC, R, H), lambda i: (0, i, 0)),
        pl.BlockSpec((R, 1), lambda i: (i, 0)),
        pl.BlockSpec((R, H), lambda i: (i, 0)),
        _w_spec(), _w_spec(), _w_spec(), _b_spec(), _b_spec(),
    ],
    out_specs=pl.BlockSpec((R, H), lambda i: (i, 0)),
    out_shape=jax.ShapeDtypeStruct((N, H), jnp.float32),
)


def _pack_bf16(x):
    xp = x.reshape(-1, 4, 2, 16).transpose(0, 1, 3, 2).reshape(-1, 128)
    xb = xp.astype(jnp.bfloat16)
    return lax.bitcast_convert_type(xb.reshape(-1, 64, 2), jnp.int32)


def kernel(x_source, x_target, edge_index_st, edge_index_ts,
           l0_st_Wl, l0_st_bl, l0_st_Wr, l0_ts_Wl, l0_ts_bl, l0_ts_Wr,
           l1_st_Wl, l1_st_bl, l1_st_Wr, l1_ts_Wl, l1_ts_bl, l1_ts_Wr,
           lin_W, lin_b):
    f32 = jnp.float32
    xp_cat = _pack_bf16(jnp.concatenate([x_source, x_target], axis=0))

    s_st = edge_index_st[0].astype(jnp.int32)
    d_st = edge_index_st[1].astype(jnp.int32)
    s_ts = edge_index_ts[0].astype(jnp.int32)
    d_ts = edge_index_ts[1].astype(jnp.int32)

    padA = EPC_A - E
    pad_src = (jnp.arange(padA, dtype=jnp.int32) * 53) % N
    pad_dst = N + jnp.arange(padA, dtype=jnp.int32) % (NP - N)
    srcA = jnp.concatenate([
        s_st, pad_src,
        s_ts + N, pad_src,
    ]).reshape(-1, 128)
    dstA = jnp.concatenate([
        d_st, pad_dst,
        d_ts, pad_dst,
    ]).reshape(-1, 128)

    zeros_big = jnp.zeros((NP, H), f32)
    agg0, cnt0 = _sc_layer0(xp_cat, srcA, dstA, zeros_big)
    cntb = cnt0[:, :N, None]

    xs1, xt1 = _tc_l0(agg0, cntb, x_source, x_target,
                      l0_st_Wl, l0_st_Wr, l0_ts_Wl, l0_ts_Wr,
                      l0_st_bl.reshape(1, H), l0_ts_bl.reshape(1, H))

    padB = EB - E
    srcB = jnp.concatenate([s_ts, pad_src]).reshape(-1, 128)
    dstB = jnp.concatenate([d_ts, pad_dst]).reshape(-1, 128)
    agg1 = _sc_layer1(_pack_bf16(xt1), srcB, dstB, zeros_big)

    return _tc_l1(agg1, cntb[1], xs1,
                  l1_ts_Wl, l1_ts_Wr, lin_W,
                  l1_ts_bl.reshape(1, H), lin_b.reshape(1, H))
